# baseline (speedup 1.0000x reference)
import functools

import jax
import jax.numpy as jnp
from jax import lax
from jax.experimental import pallas as pl
from jax.experimental.pallas import tpu as pltpu
from jax.experimental.pallas import tpu_sc as plsc

_NUM_CORES = 2
_NUM_SUBCORES = 16
_NUM_WORKERS = _NUM_CORES * _NUM_SUBCORES

_BS_PER_CHUNK = 4
_NBUF = 4


def _gather_kernel(n_chunks, chunk, bpc, ids_hbm, table_hbm, out_hbm,
                   idx_v, rows_v, gsems, wsems):
  wid = lax.axis_index("s") * _NUM_CORES + lax.axis_index("c")
  rows_per_worker = n_chunks * chunk
  row_base = wid * rows_per_worker
  b_base = wid * n_chunks * bpc
  n_groups = n_chunks // _NBUF

  pltpu.sync_copy(ids_hbm.at[pl.ds(b_base, n_chunks * bpc)], idx_v)

  def start_gather(j, b):
    for k in range(bpc):
      pltpu.async_copy(table_hbm.at[idx_v.at[j * bpc + k]],
                       rows_v.at[b, k], gsems[b])

  def wait_gather(j, b):
    for k in range(bpc):
      pltpu.make_async_copy(table_hbm.at[idx_v.at[j * bpc + k]],
                            rows_v.at[b, k], gsems[b]).wait()

  def out_slice(j):
    return out_hbm.at[pl.ds(b_base + j * bpc, bpc)]

  def start_write(j, b):
    pltpu.async_copy(rows_v.at[b], out_slice(j), wsems[b])

  def wait_write(j, b):
    pltpu.make_async_copy(rows_v.at[b], out_slice(j), wsems[b]).wait()

  for b in range(_NBUF):
    start_gather(b, b)

  @pl.loop(0, n_groups - 1)
  def _(g):
    first = g * _NBUF
    for b in range(_NBUF):
      wait_gather(first + b, b)
      start_write(first + b, b)
    for b in range(_NBUF):
      wait_write(first + b, b)
      start_gather(first + _NBUF + b, b)

  last = (n_groups - 1) * _NBUF
  for b in range(_NBUF):
    wait_gather(last + b, b)
    start_write(last + b, b)
  for b in range(_NBUF):
    wait_write(last + b, b)


def kernel(relation_ids, table):
  batch, hist = relation_ids.shape
  vocab, dim = table.shape
  total = batch * hist
  chunk = _BS_PER_CHUNK * hist
  assert batch % (_NUM_WORKERS * _BS_PER_CHUNK * _NBUF) == 0
  rows_per_worker = total // _NUM_WORKERS
  n_chunks = rows_per_worker // chunk

  ids = relation_ids.astype(jnp.int32)

  mesh = plsc.VectorSubcoreMesh(core_axis_name="c", subcore_axis_name="s")
  grab = pl.kernel(
      functools.partial(_gather_kernel, n_chunks, chunk, _BS_PER_CHUNK),
      out_type=jax.ShapeDtypeStruct((batch, hist, dim), jnp.float32),
      mesh=mesh,
      scratch_types=[
          pltpu.VMEM((n_chunks * _BS_PER_CHUNK, hist), jnp.int32),
          pltpu.VMEM((_NBUF, _BS_PER_CHUNK, hist, dim), jnp.float32),
          [pltpu.SemaphoreType.DMA] * _NBUF,
          [pltpu.SemaphoreType.DMA] * _NBUF,
      ],
      compiler_params=pltpu.CompilerParams(use_tc_tiling_on_sc=False),
  )
  return grab(ids, table)


# speedup vs baseline: 1.7876x; 1.7876x over previous
import functools

import jax
import jax.numpy as jnp
from jax import lax
from jax.experimental import pallas as pl
from jax.experimental.pallas import tpu as pltpu
from jax.experimental.pallas import tpu_sc as plsc

_NUM_CORES = 2
_NUM_SUBCORES = 16
_NUM_WORKERS = _NUM_CORES * _NUM_SUBCORES

_BS_PER_CHUNK = 4
_NBUF = 4


def _gather_kernel(n_chunks, bpc, hist, hist_pad, ids_hbm, table_hbm, out_hbm,
                   idx_v, rows_v, gsems, wsems):
  wid = lax.axis_index("s") * _NUM_CORES + lax.axis_index("c")
  bs_per_worker = n_chunks * bpc
  b_base = wid * bs_per_worker
  dim = table_hbm.shape[-1]

  pltpu.sync_copy(ids_hbm.at[pl.ds(b_base, bs_per_worker)], idx_v)

  def start_gather(j, b):
    for k in range(bpc):
      pltpu.async_copy(table_hbm.at[idx_v.at[j * bpc + k]],
                       rows_v.at[b, k], gsems[b])

  def wait_gather(j, b):
    for k in range(bpc):
      pltpu.make_async_copy(table_hbm.at[idx_v.at[j * bpc + k]],
                            rows_v.at[b, k], gsems[b]).wait()

  def out_slice(j, k):
    row0 = (b_base + j * bpc + k) * hist_pad
    return out_hbm.at[pl.ds(row0, hist), pl.ds(0, dim)]

  def start_write(j, b):
    for k in range(bpc):
      pltpu.async_copy(rows_v.at[b, k], out_slice(j, k), wsems[b])

  def wait_write(j, b):
    for k in range(bpc):
      pltpu.make_async_copy(rows_v.at[b, k], out_slice(j, k), wsems[b]).wait()

  for b in range(_NBUF):
    start_gather(b, b)

  n_groups = n_chunks // _NBUF

  @pl.loop(0, n_groups - 1)
  def _(g):
    first = g * _NBUF
    for b in range(_NBUF):
      wait_gather(first + b, b)
      start_write(first + b, b)
    for b in range(_NBUF):
      wait_write(first + b, b)
      start_gather(first + _NBUF + b, b)

  last = (n_groups - 1) * _NBUF
  for b in range(_NBUF):
    wait_gather(last + b, b)
    start_write(last + b, b)
  for b in range(_NBUF):
    wait_write(last + b, b)


def kernel(relation_ids, table):
  batch, hist = relation_ids.shape
  vocab, dim = table.shape
  hist_pad = (hist + 7) // 8 * 8
  dim_pad = 128
  assert batch % (_NUM_WORKERS * _BS_PER_CHUNK * _NBUF) == 0
  bs_per_worker = batch // _NUM_WORKERS
  n_chunks = bs_per_worker // _BS_PER_CHUNK

  ids = relation_ids.astype(jnp.int32)

  mesh = plsc.VectorSubcoreMesh(core_axis_name="c", subcore_axis_name="s")
  grab = pl.kernel(
      functools.partial(_gather_kernel, n_chunks, _BS_PER_CHUNK, hist,
                        hist_pad),
      out_type=jax.ShapeDtypeStruct((batch * hist_pad, dim_pad), jnp.float32),
      mesh=mesh,
      scratch_types=[
          pltpu.VMEM((bs_per_worker, hist), jnp.int32),
          pltpu.VMEM((_NBUF, _BS_PER_CHUNK, hist, dim), jnp.float32),
          [pltpu.SemaphoreType.DMA] * _NBUF,
          [pltpu.SemaphoreType.DMA] * _NBUF,
      ],
      compiler_params=pltpu.CompilerParams(use_tc_tiling_on_sc=False),
  )
  out = grab(ids, table)
  return out.reshape(batch, hist_pad, dim_pad)[:, :hist, :dim]


# speedup vs baseline: 1.7878x; 1.0001x over previous
import functools

import jax
import jax.numpy as jnp
from jax import lax
from jax.experimental import pallas as pl
from jax.experimental.pallas import tpu as pltpu
from jax.experimental.pallas import tpu_sc as plsc

_NUM_CORES = 2
_NUM_SUBCORES = 16
_NUM_WORKERS = _NUM_CORES * _NUM_SUBCORES

_BS_PER_CHUNK = 8
_NBUF = 4


def _gather_kernel(n_chunks, bpc, hist, hist_pad, ids_hbm, table_hbm, out_hbm,
                   idx_v, rows_v, gsems, wsems):
  wid = lax.axis_index("s") * _NUM_CORES + lax.axis_index("c")
  bs_per_worker = n_chunks * bpc
  b_base = wid * bs_per_worker
  dim = table_hbm.shape[-1]

  pltpu.sync_copy(ids_hbm.at[pl.ds(b_base, bs_per_worker)], idx_v)

  def start_gather(j, b):
    for k in range(bpc):
      pltpu.async_copy(table_hbm.at[idx_v.at[j * bpc + k]],
                       rows_v.at[b, k], gsems[b])

  def wait_gather(j, b):
    for k in range(bpc):
      pltpu.make_async_copy(table_hbm.at[idx_v.at[j * bpc + k]],
                            rows_v.at[b, k], gsems[b]).wait()

  def out_slice(j, k):
    row0 = (b_base + j * bpc + k) * hist_pad
    return out_hbm.at[pl.ds(row0, hist), pl.ds(0, dim)]

  def start_write(j, b):
    for k in range(bpc):
      pltpu.async_copy(rows_v.at[b, k], out_slice(j, k), wsems[b])

  def wait_write(j, b):
    for k in range(bpc):
      pltpu.make_async_copy(rows_v.at[b, k], out_slice(j, k), wsems[b]).wait()

  for b in range(_NBUF):
    start_gather(b, b)

  n_groups = n_chunks // _NBUF

  @pl.loop(0, n_groups - 1)
  def _(g):
    first = g * _NBUF
    for b in range(_NBUF):
      wait_gather(first + b, b)
      start_write(first + b, b)
    for b in range(_NBUF):
      wait_write(first + b, b)
      start_gather(first + _NBUF + b, b)

  last = (n_groups - 1) * _NBUF
  for b in range(_NBUF):
    wait_gather(last + b, b)
    start_write(last + b, b)
  for b in range(_NBUF):
    wait_write(last + b, b)


def kernel(relation_ids, table):
  batch, hist = relation_ids.shape
  vocab, dim = table.shape
  hist_pad = (hist + 7) // 8 * 8
  dim_pad = 128
  assert batch % (_NUM_WORKERS * _BS_PER_CHUNK * _NBUF) == 0
  bs_per_worker = batch // _NUM_WORKERS
  n_chunks = bs_per_worker // _BS_PER_CHUNK

  ids = relation_ids.astype(jnp.int32)

  mesh = plsc.VectorSubcoreMesh(core_axis_name="c", subcore_axis_name="s")
  grab = pl.kernel(
      functools.partial(_gather_kernel, n_chunks, _BS_PER_CHUNK, hist,
                        hist_pad),
      out_type=jax.ShapeDtypeStruct((batch * hist_pad, dim_pad), jnp.float32),
      mesh=mesh,
      scratch_types=[
          pltpu.VMEM((bs_per_worker, hist), jnp.int32),
          pltpu.VMEM((_NBUF, _BS_PER_CHUNK, hist, dim), jnp.float32),
          [pltpu.SemaphoreType.DMA] * _NBUF,
          [pltpu.SemaphoreType.DMA] * _NBUF,
      ],
      compiler_params=pltpu.CompilerParams(use_tc_tiling_on_sc=False),
  )
  out = grab(ids, table)
  return out.reshape(batch, hist_pad, dim_pad)[:, :hist, :dim]
